# baseline (speedup 1.0000x reference)
import functools

import jax
import jax.numpy as jnp
from jax import lax
from jax.experimental import pallas as pl
from jax.experimental.pallas import tpu as pltpu
from jax.experimental.pallas import tpu_sc as plsc

_B = 16384
_D = 32
_NC = 2
_NS = 16
_NW = _NC * _NS
_BPW = _B // _NW
_CHUNK = 128
_NCHUNK = _BPW // _CHUNK
_LANES = 16


def _gmf_body(user_hbm, item_hbm, ut_hbm, it_hbm, out_hbm,
              uidx, iidx, urows, irows, sem):
    wid = lax.axis_index("s") * _NC + lax.axis_index("c")
    base = wid * _BPW

    pltpu.sync_copy(user_hbm.at[wid], uidx)
    pltpu.sync_copy(item_hbm.at[wid], iidx)

    copies = []
    for j in range(_NCHUNK):
        dst = urows.at[pl.ds(j * _CHUNK, _CHUNK)]
        copies.append(pltpu.async_copy(ut_hbm.at[uidx.at[j]], dst, sem))
        dst = irows.at[pl.ds(j * _CHUNK, _CHUNK)]
        copies.append(pltpu.async_copy(it_hbm.at[iidx.at[j]], dst, sem))
    for c in copies:
        c.wait()

    def body(k, _):
        for r in range(4):
            i = k * 4 + r
            for c in range(0, _D, _LANES):
                u = urows[i, pl.ds(c, _LANES)]
                v = irows[i, pl.ds(c, _LANES)]
                urows[i, pl.ds(c, _LANES)] = u * v
        return 0

    lax.fori_loop(0, _BPW // 4, body, 0)

    pltpu.sync_copy(urows, out_hbm.at[pl.ds(base, _BPW)])


@functools.partial(
    pl.kernel,
    out_type=jax.ShapeDtypeStruct((_B, _D), jnp.float32),
    mesh=plsc.VectorSubcoreMesh(core_axis_name="c", subcore_axis_name="s"),
    compiler_params=pltpu.CompilerParams(use_tc_tiling_on_sc=False),
    scratch_types=[
        pltpu.VMEM((_NCHUNK, _CHUNK), jnp.int32),
        pltpu.VMEM((_NCHUNK, _CHUNK), jnp.int32),
        pltpu.VMEM((_BPW, _D), jnp.float32),
        pltpu.VMEM((_BPW, _D), jnp.float32),
        pltpu.SemaphoreType.DMA,
    ],
)
def _gmf(user_hbm, item_hbm, ut_hbm, it_hbm, out_hbm,
         uidx, iidx, urows, irows, sem):
    _gmf_body(user_hbm, item_hbm, ut_hbm, it_hbm, out_hbm,
              uidx, iidx, urows, irows, sem)


def kernel(user, item, user_table, item_table):
    user = user.astype(jnp.int32).reshape(_NW, _NCHUNK, _CHUNK)
    item = item.astype(jnp.int32).reshape(_NW, _NCHUNK, _CHUNK)
    return _gmf(user, item, user_table, item_table)
